# baseline (speedup 1.0000x reference)
import functools

import jax
import jax.numpy as jnp
from jax import lax
from jax.experimental import pallas as pl
from jax.experimental.pallas import tpu as pltpu
from jax.experimental.pallas import tpu_sc as plsc

_B, _L = 4096, 200
_V = _L // 2
_NC, _NS = 2, 16
_NW = _NC * _NS
_HB = _B // 2
_RPW = _HB // _NW
_CW = 128
_VW = _RPW * _V
_LANES = 16
_CHUNKS = -(-_V // _LANES)


def _make_sc_vals(half):
    def body(dense_hbm, vals_hbm, vbuf, cbuf):
        c = lax.axis_index("c")
        s = lax.axis_index("s")
        wid = s * _NC + c
        rbase = half * _HB + wid * _RPW

        pltpu.sync_copy(dense_hbm.at[pl.ds(rbase, _RPW), pl.ds(0, _CW)], vbuf)

        def crow(i, carry):
            for j in range(_CHUNKS):
                cbuf[pl.ds(i * _V + j * _LANES, _LANES)] = (
                    vbuf[i, pl.ds(j * _LANES, _LANES)])
            return carry

        lax.fori_loop(0, _RPW, crow, 0)
        pltpu.sync_copy(cbuf.at[pl.ds(0, _VW)],
                        vals_hbm.at[pl.ds(wid * _VW, _VW)])

    return pl.kernel(
        body,
        out_type=jax.ShapeDtypeStruct((_HB * _V,), jnp.float32),
        mesh=plsc.VectorSubcoreMesh(core_axis_name="c", subcore_axis_name="s"),
        scratch_types=[pltpu.VMEM((_RPW, _CW), jnp.float32),
                       pltpu.VMEM((_VW + _CHUNKS * _LANES - _V,), jnp.float32)],
    )


_sc_vals0 = _make_sc_vals(0)
_sc_vals1 = _make_sc_vals(1)

_IDX_BLK = 12800


def _tc_idx_body(o_ref):
    rbase = pl.program_id(0) * (_IDX_BLK // _V)
    p = lax.broadcasted_iota(jnp.int32, (_IDX_BLK, 2), 0)
    j = lax.broadcasted_iota(jnp.int32, (_IDX_BLK, 2), 1)
    q = (p.astype(jnp.float32) * jnp.float32(1.0 / _V)).astype(jnp.int32)
    rem = p - q * _V
    over = (rem >= _V).astype(jnp.int32)
    q = q + over
    rem = rem - _V * over
    under = (rem < 0).astype(jnp.int32)
    q = q - under
    rem = rem + _V * under
    o_ref[...] = jnp.where(j == 0, rbase + q, rem)


_tc_idx = pl.pallas_call(
    _tc_idx_body,
    out_shape=jax.ShapeDtypeStruct((_B * _V, 2), jnp.int32),
    grid=(_B * _V // _IDX_BLK,),
    out_specs=pl.BlockSpec((_IDX_BLK, 2), lambda b: (b, 0)),
)


def kernel(dense_tensor):
    b, l = dense_tensor.shape
    v0 = _sc_vals0(dense_tensor)
    v1 = _sc_vals1(dense_tensor)
    weight_vals = jnp.concatenate([v0, v1])
    weight_idx = _tc_idx().astype(jnp.int64)
    dense_shape = jnp.array([b, l], dtype=jnp.int64)
    return weight_idx, weight_vals, dense_shape
